# Initial kernel scaffold; baseline (speedup 1.0000x reference)
import jax
import jax.numpy as jnp
from jax.experimental import pallas as pl


def kernel(x, edge_index, edge_attr, lin_w, lin_b, w0, b0, w1, b1):
    raise NotImplementedError("write your pallas kernel here")


# speedup vs baseline: 2.8667x; 2.8667x over previous
import functools
import jax
import jax.numpy as jnp
from jax import lax
from jax.experimental import pallas as pl
from jax.experimental.pallas import tpu as pltpu
from jax.experimental.pallas import tpu_sc as plsc

N_NODES = 10000
D = 128
E = 320000
HID = 64

NC, NS = 2, 16
NW = NC * NS
EPW = E // NW
BLK = 128
NFULL = EPW // BLK
TAIL = EPW - NFULL * BLK
CH = 80
NCHUNK = N_NODES // CH


def _emb_body(attr_ref, w_ref, b_ref, out_ref):
    out_ref[...] = (
        jnp.dot(attr_ref[...], w_ref[...], preferred_element_type=jnp.float32)
        + b_ref[...]
    )


def _edge_emb(edge_attr, w16x128, b1x128):
    rb = 3200
    return pl.pallas_call(
        _emb_body,
        grid=(E // rb,),
        in_specs=[
            pl.BlockSpec((rb, 16), lambda i: (i, 0)),
            pl.BlockSpec((16, D), lambda i: (0, 0)),
            pl.BlockSpec((1, D), lambda i: (0, 0)),
        ],
        out_specs=pl.BlockSpec((rb, D), lambda i: (i, 0)),
        out_shape=jax.ShapeDtypeStruct((E, D), jnp.float32),
    )(edge_attr, w16x128, b1x128)


def _sc_body(x_hbm, emb_hbm, src_hbm, dst_hbm, out_hbm,
             isrc, idst, rows, embv, tsrc, tdst, trows, tembv, zbuf,
             accum, sem):
    cid = lax.axis_index("c")
    sid = lax.axis_index("s")
    wid = sid * NC + cid

    zv = jnp.zeros((16,), jnp.float32)

    def zero_body(i, _):
        for j in range(D // 16):
            zbuf[i, pl.ds(j * 16, 16)] = zv
        return 0

    lax.fori_loop(0, CH, zero_body, 0)
    nch = jnp.where(sid < NCHUNK - (NCHUNK // NS) * NS, NCHUNK // NS + 1,
                    NCHUNK // NS)

    def zchunk(k, _):
        r0 = pl.multiple_of((sid + k * NS) * CH, 8)
        pltpu.sync_copy(zbuf, accum.at[pl.ds(r0, CH)])
        return 0

    lax.fori_loop(0, nch, zchunk, 0)
    plsc.subcore_barrier()

    def process(base, n, isrc_b, idst_b, rows_b, emb_b):
        pltpu.sync_copy(src_hbm.at[pl.ds(base, n)], isrc_b)
        pltpu.sync_copy(dst_hbm.at[pl.ds(base, n)], idst_b)
        pltpu.sync_copy(emb_hbm.at[pl.ds(base, n)], emb_b)
        pltpu.async_copy(x_hbm.at[isrc_b], rows_b, sem).wait()

        def edge_body(i, _):
            for j in range(D // 16):
                sl = pl.ds(j * 16, 16)
                rows_b[i, sl] = jnp.maximum(rows_b[i, sl] + emb_b[i, sl], 0.0)
            return 0

        lax.fori_loop(0, n, edge_body, 0)
        pltpu.sync_copy(rows_b, accum.at[idst_b], add=True)

    base_w = wid * EPW

    def blk_body(b, _):
        process(pl.multiple_of(base_w + b * BLK, 8), BLK, isrc, idst, rows, embv)
        return 0

    lax.fori_loop(0, NFULL, blk_body, 0)
    process(base_w + NFULL * BLK, TAIL, tsrc, tdst, trows, tembv)

    plsc.subcore_barrier()

    def wchunk(k, _):
        r0 = pl.multiple_of((sid + k * NS) * CH, 8)
        pltpu.sync_copy(accum.at[pl.ds(r0, CH)], out_hbm.at[cid, pl.ds(r0, CH)])
        return 0

    lax.fori_loop(0, nch, wchunk, 0)


def _sc_aggregate(x, emb, src, dst):
    mesh = plsc.VectorSubcoreMesh(core_axis_name="c", subcore_axis_name="s")
    f = pl.kernel(
        _sc_body,
        out_type=jax.ShapeDtypeStruct((NC, N_NODES, D), jnp.float32),
        mesh=mesh,
        scratch_types=[
            pltpu.VMEM((BLK,), jnp.int32),
            pltpu.VMEM((BLK,), jnp.int32),
            pltpu.VMEM((BLK, D), jnp.float32),
            pltpu.VMEM((BLK, D), jnp.float32),
            pltpu.VMEM((TAIL,), jnp.int32),
            pltpu.VMEM((TAIL,), jnp.int32),
            pltpu.VMEM((TAIL, D), jnp.float32),
            pltpu.VMEM((TAIL, D), jnp.float32),
            pltpu.VMEM((CH, D), jnp.float32),
            pltpu.VMEM_SHARED((N_NODES, D), jnp.float32),
            pltpu.SemaphoreType.DMA,
        ],
    )
    return f(x, emb, src, dst)


def _mlp_body(x_ref, p0_ref, p1_ref, w0_ref, b0_ref, w1_ref, b1_ref, out_ref):
    h = x_ref[...] + p0_ref[...] + p1_ref[...]
    h = jnp.maximum(
        jnp.dot(h, w0_ref[...], preferred_element_type=jnp.float32) + b0_ref[...],
        0.0,
    )
    out_ref[...] = (
        jnp.dot(h, w1_ref[...], preferred_element_type=jnp.float32) + b1_ref[...]
    )


def _mlp(x, p0, p1, w0t, b0r, w1t, b1r):
    rb = 2000
    return pl.pallas_call(
        _mlp_body,
        grid=(N_NODES // rb,),
        in_specs=[
            pl.BlockSpec((rb, D), lambda i: (i, 0)),
            pl.BlockSpec((rb, D), lambda i: (i, 0)),
            pl.BlockSpec((rb, D), lambda i: (i, 0)),
            pl.BlockSpec((D, HID), lambda i: (0, 0)),
            pl.BlockSpec((1, HID), lambda i: (0, 0)),
            pl.BlockSpec((HID, D), lambda i: (0, 0)),
            pl.BlockSpec((1, D), lambda i: (0, 0)),
        ],
        out_specs=pl.BlockSpec((rb, D), lambda i: (i, 0)),
        out_shape=jax.ShapeDtypeStruct((N_NODES, D), jnp.float32),
    )(x, p0, p1, w0t, b0r, w1t, b1r)


def kernel(x, edge_index, edge_attr, lin_w, lin_b, w0, b0, w1, b1):
    src = edge_index[0].astype(jnp.int32)
    dst = edge_index[1].astype(jnp.int32)
    emb = _edge_emb(edge_attr, lin_w.T, lin_b.reshape(1, D))
    partials = _sc_aggregate(x, emb, src, dst)
    return _mlp(x, partials[0], partials[1],
                w0.T, b0.reshape(1, HID), w1.T, b1.reshape(1, D))


# speedup vs baseline: 4.3662x; 1.5231x over previous
import functools
import jax
import jax.numpy as jnp
from jax import lax
from jax.experimental import pallas as pl
from jax.experimental.pallas import tpu as pltpu
from jax.experimental.pallas import tpu_sc as plsc

N_NODES = 10000
D = 128
E = 320000
HID = 64

NC, NS = 2, 16
NW = NC * NS
BLK = 64
NB = E // BLK
NBW = NB // NW
NXTRA = NB - NBW * NW
UNROLL = 12
NSUP = NBW // UNROLL
CH = 40
NCHUNK = N_NODES // CH


def _emb_body(attr_ref, w_ref, b_ref, out_ref):
    out_ref[...] = (
        jnp.dot(attr_ref[...], w_ref[...], preferred_element_type=jnp.float32)
        + b_ref[...]
    )


def _edge_emb(edge_attr, w16x128, b1x128):
    rb = 3200
    return pl.pallas_call(
        _emb_body,
        grid=(E // rb,),
        in_specs=[
            pl.BlockSpec((rb, 16), lambda i: (i, 0)),
            pl.BlockSpec((16, D), lambda i: (0, 0)),
            pl.BlockSpec((1, D), lambda i: (0, 0)),
        ],
        out_specs=pl.BlockSpec((rb, D), lambda i: (i, 0)),
        out_shape=jax.ShapeDtypeStruct((E, D), jnp.float32),
    )(edge_attr, w16x128, b1x128)


def _sc_body(x_hbm, emb_hbm, src2_hbm, dst2_hbm, out_hbm,
             rows0, rows1, rows2, emb0, emb1,
             is0, is1, is2, id0, id1, id2, id3,
             accum,
             sg0, sg1, sg2, se0, se1, ss0, ss1, si0, si1, si2):
    cid = lax.axis_index("c")
    sid = lax.axis_index("s")
    wid = sid * NC + cid
    rows = (rows0, rows1, rows2)
    embv = (emb0, emb1)
    isrc = (is0, is1, is2)
    idst = (id0, id1, id2, id3)
    sg = (sg0, sg1, sg2)
    se = (se0, se1)
    ss = (ss0, ss1)
    si = (si0, si1, si2)

    zv = jnp.zeros((16,), jnp.float32)

    def zero_body(i, _):
        for j in range(D // 16):
            rows0[i, pl.ds(j * 16, 16)] = zv
        return 0

    lax.fori_loop(0, CH, zero_body, 0)
    nch = jnp.where(sid < NCHUNK - (NCHUNK // NS) * NS, NCHUNK // NS + 1,
                    NCHUNK // NS)

    def zchunk(k, _):
        r0 = pl.multiple_of((sid + k * NS) * CH, 8)
        pltpu.sync_copy(rows0.at[pl.ds(0, CH)], accum.at[pl.ds(r0, CH)])
        return 0

    lax.fori_loop(0, nch, zchunk, 0)
    plsc.subcore_barrier()


    def fire_idx(t, k3, k4):
        blk = t * NW + wid
        pltpu.async_copy(src2_hbm.at[blk], isrc[k3], si[k3])
        pltpu.async_copy(dst2_hbm.at[blk], idst[k4], si[k3])

    def drain_idx(k3):
        pltpu.make_async_copy(src2_hbm.at[0], isrc[k3], si[k3]).wait()
        pltpu.make_async_copy(dst2_hbm.at[0], idst[0], si[k3]).wait()

    def ebase(t):
        return pl.multiple_of((t * NW + wid) * BLK, 8)

    def fire_gather(t, k3, k2):
        pltpu.async_copy(x_hbm.at[isrc[k3]], rows[k3], sg[k3])
        pltpu.async_copy(emb_hbm.at[pl.ds(ebase(t), BLK)], embv[k2], se[k2])

    def wait_gather(t, k3, k2):
        pltpu.make_async_copy(x_hbm.at[isrc[k3]], rows[k3], sg[k3]).wait()
        pltpu.make_async_copy(emb_hbm.at[pl.ds(ebase(t), BLK)], embv[k2],
                              se[k2]).wait()

    def compute(k3, k2):
        def body(i, _):
            for j in range(D // 16):
                sl = pl.ds(j * 16, 16)
                rows[k3][i, sl] = jnp.maximum(
                    rows[k3][i, sl] + embv[k2][i, sl], 0.0)
            return 0

        lax.fori_loop(0, BLK, body, 0)

    fire_idx(0, 0, 0)
    fire_idx(1, 1, 1)
    fire_idx(2, 2, 2)
    drain_idx(0)
    fire_gather(0, 0, 0)
    drain_idx(1)
    fire_gather(1, 1, 1)

    def super_body(s, _):
        for u in range(UNROLL):
            t = s * UNROLL + u
            k3, k2, k4 = u % 3, u % 2, u % 4
            wait_gather(t, k3, k2)
            compute(k3, k2)
            pltpu.async_copy(rows[k3], accum.at[idst[k4]], ss[u % 2], add=True)

            def drain_scat():
                pltpu.make_async_copy(rows[(u - 1) % 3],
                                      accum.at[idst[(u - 1) % 4]],
                                      ss[(u - 1) % 2]).wait()

            if u == 0:
                pl.when(s > 0)(drain_scat)
            else:
                drain_scat()

            @pl.when(t + 2 < NBW)
            def _():
                drain_idx((u + 2) % 3)
                fire_gather(t + 2, (u + 2) % 3, k2)

            @pl.when(t + 3 < NBW)
            def _():
                fire_idx(t + 3, u % 3, (u + 3) % 4)
        return 0

    lax.fori_loop(0, NSUP, super_body, 0)
    pltpu.make_async_copy(rows[(NBW - 1) % 3], accum.at[idst[(NBW - 1) % 4]],
                          ss[(NBW - 1) % 2]).wait()

    @pl.when(wid < NXTRA)
    def _():
        blk = NBW * NW + wid
        pltpu.sync_copy(src2_hbm.at[blk], isrc[0])
        pltpu.sync_copy(dst2_hbm.at[blk], idst[0])
        pltpu.async_copy(x_hbm.at[isrc[0]], rows[0], sg[0]).wait()
        pltpu.sync_copy(emb_hbm.at[pl.ds(ebase(NBW), BLK)], embv[0])
        compute(0, 0)
        pltpu.sync_copy(rows[0], accum.at[idst[0]], add=True)

    plsc.subcore_barrier()

    def wchunk(k, _):
        r0 = pl.multiple_of((sid + k * NS) * CH, 8)
        pltpu.sync_copy(accum.at[pl.ds(r0, CH)], out_hbm.at[cid, pl.ds(r0, CH)])
        return 0

    lax.fori_loop(0, nch, wchunk, 0)


def _sc_aggregate(x, emb, src2, dst2):
    mesh = plsc.VectorSubcoreMesh(core_axis_name="c", subcore_axis_name="s")
    f = pl.kernel(
        _sc_body,
        out_type=jax.ShapeDtypeStruct((NC, N_NODES, D), jnp.float32),
        mesh=mesh,
        scratch_types=(
            [pltpu.VMEM((BLK, D), jnp.float32)] * 3
            + [pltpu.VMEM((BLK, D), jnp.float32)] * 2
            + [pltpu.VMEM((BLK,), jnp.int32)] * 3
            + [pltpu.VMEM((BLK,), jnp.int32)] * 4
            + [pltpu.VMEM_SHARED((N_NODES, D), jnp.float32)]
            + [pltpu.SemaphoreType.DMA] * 10
        ),
    )
    return f(x, emb, src2, dst2)


def _mlp_body(x_ref, p0_ref, p1_ref, w0_ref, b0_ref, w1_ref, b1_ref, out_ref):
    h = x_ref[...] + p0_ref[...] + p1_ref[...]
    h = jnp.maximum(
        jnp.dot(h, w0_ref[...], preferred_element_type=jnp.float32) + b0_ref[...],
        0.0,
    )
    out_ref[...] = (
        jnp.dot(h, w1_ref[...], preferred_element_type=jnp.float32) + b1_ref[...]
    )


def _mlp(x, p0, p1, w0t, b0r, w1t, b1r):
    rb = 2000
    return pl.pallas_call(
        _mlp_body,
        grid=(N_NODES // rb,),
        in_specs=[
            pl.BlockSpec((rb, D), lambda i: (i, 0)),
            pl.BlockSpec((rb, D), lambda i: (i, 0)),
            pl.BlockSpec((rb, D), lambda i: (i, 0)),
            pl.BlockSpec((D, HID), lambda i: (0, 0)),
            pl.BlockSpec((1, HID), lambda i: (0, 0)),
            pl.BlockSpec((HID, D), lambda i: (0, 0)),
            pl.BlockSpec((1, D), lambda i: (0, 0)),
        ],
        out_specs=pl.BlockSpec((rb, D), lambda i: (i, 0)),
        out_shape=jax.ShapeDtypeStruct((N_NODES, D), jnp.float32),
    )(x, p0, p1, w0t, b0r, w1t, b1r)


def kernel(x, edge_index, edge_attr, lin_w, lin_b, w0, b0, w1, b1):
    src2 = edge_index[0].astype(jnp.int32).reshape(NB, BLK)
    dst2 = edge_index[1].astype(jnp.int32).reshape(NB, BLK)
    emb = _edge_emb(edge_attr, lin_w.T, lin_b.reshape(1, D))
    partials = _sc_aggregate(x, emb, src2, dst2)
    return _mlp(x, partials[0], partials[1],
                w0.T, b0.reshape(1, HID), w1.T, b1.reshape(1, D))
